# Initial kernel scaffold; baseline (speedup 1.0000x reference)
import jax
import jax.numpy as jnp
from jax.experimental import pallas as pl


def kernel(et, mp, co, vol, comp_idx, Wq, bq, Wk, bk, Wv, bv, Wo, bo):
    raise NotImplementedError("write your pallas kernel here")


# speedup vs baseline: 4.2373x; 4.2373x over previous
import functools
import math

import jax
import jax.numpy as jnp
import numpy as np
from jax.experimental import pallas as pl
from jax.experimental.pallas import tpu as pltpu

_SEQ_LEN = 512
_D_MODEL = 256
_H = 4
_FACTOR = 4
_SZ = 4
_B = _SZ
_L = _SEQ_LEN * 4
_E = _D_MODEL // _H
_U = _FACTOR * int(np.ceil(np.log(_L)))
_NEG = -1e30

_idx_sample = np.asarray(
    jax.random.randint(jax.random.key(42), (_L, _U), 0, _L)
)

_C = np.zeros((_L, _L), dtype=np.int32)
np.add.at(_C, (np.arange(_L)[:, None], _idx_sample), 1)

_a = np.arange(_L)
_inv = (_a % 512) * 4 + _a // 512
_CpT = _C.T[_inv][:, _inv].astype(np.int8)


def _attn_body(cc_ref, wq_ref, bq_ref, wk_ref, bk_ref, wv_ref, bv_ref,
               wo_ref, bo_ref, cpt_ref, out_ref):
    f32 = jnp.float32
    ccb = cc_ref[...]

    def proj(w_ref, b_ref):
        m = jax.lax.dot_general(ccb, w_ref[...], (((1,), (1,)), ((), ())))
        m = m + b_ref[...]
        return jnp.concatenate(
            [m[:, 64 * p:64 * (p + 1)] for p in range(4)], axis=0)

    qp = proj(wq_ref, bq_ref)
    kp = proj(wk_ref, bk_ref)
    vp = proj(wv_ref, bv_ref)

    rows = []
    for lb in range(16):
        qb = qp[128 * lb:128 * (lb + 1), :]
        qk = jax.lax.dot_general(kp, qb, (((1,), (1,)), ((), ())))
        cb = cpt_ref[:, 128 * lb:128 * (lb + 1)].astype(f32)
        mx = jnp.max(jnp.where(cb > 0.0, qk, _NEG), axis=0, keepdims=True)
        sm = jnp.sum(qk * cb, axis=0, keepdims=True)
        rows.append(mx - sm * (1.0 / _L))
    m_val = jnp.concatenate(rows, axis=0)

    flat = (jax.lax.broadcasted_iota(jnp.int32, (16, 128), 0) * 128
            + jax.lax.broadcasted_iota(jnp.int32, (16, 128), 1))
    lane32 = jax.lax.broadcasted_iota(jnp.int32, (1, _U), 1)

    def topk_step(it, carry):
        m_cur, fiv = carry
        mv = jnp.max(m_cur)
        fi = jnp.min(jnp.where(m_cur == mv, flat, _L))
        fiv = jnp.where(lane32 == it, fi, fiv)
        m_cur = jnp.where(flat == fi, _NEG, m_cur)
        return m_cur, fiv

    _, fiv = jax.lax.fori_loop(
        0, _U, topk_step, (m_val, jnp.full((1, _U), -1, jnp.int32)))

    row_iota = jax.lax.broadcasted_iota(jnp.int32, (_L, _U), 0)
    oht = (row_iota == fiv).astype(f32)

    hi = jax.lax.Precision.HIGHEST
    q_red = jax.lax.dot_general(oht, qp, (((0,), (0,)), ((), ())),
                                precision=hi)
    scores = jax.lax.dot_general(q_red, kp, (((1,), (1,)), ((), ())))
    scores = scores * (1.0 / math.sqrt(_E))
    smax = jnp.max(scores, axis=1, keepdims=True)
    sexp = jnp.exp(scores - smax)
    attn = sexp / jnp.sum(sexp, axis=1, keepdims=True)
    upd = jnp.dot(attn, vp)

    vsum = jnp.sum(vp, axis=0, keepdims=True)
    ctx = jnp.dot(oht, upd - vsum, precision=hi) + vsum
    ctx2d = jnp.concatenate(
        [ctx[512 * p:512 * (p + 1), :] for p in range(4)], axis=1)
    out = jax.lax.dot_general(ctx2d, wo_ref[...], (((1,), (1,)), ((), ())))
    out_ref[...] = out + bo_ref[...]


@jax.jit
def _run(cc2d, Wq, bq, Wk, bk, Wv, bv, Wo, bo):
    cpt = jnp.asarray(_CpT)
    full = lambda shape: pl.BlockSpec(shape, lambda i: (0, 0))
    out2d = pl.pallas_call(
        _attn_body,
        grid=(_B * _H,),
        in_specs=[
            pl.BlockSpec((512, 256), lambda i: (i, 0)),
            full((256, 256)), full((1, 256)),
            full((256, 256)), full((1, 256)),
            full((256, 256)), full((1, 256)),
            full((256, 256)), full((1, 256)),
            full((_L, _L)),
        ],
        out_specs=pl.BlockSpec((512, 256), lambda i: (i, 0)),
        out_shape=jax.ShapeDtypeStruct((_B * _H * 512, 256), jnp.float32),
    )(cc2d, Wq, bq.reshape(1, -1), Wk, bk.reshape(1, -1),
      Wv, bv.reshape(1, -1), Wo, bo.reshape(1, -1), cpt)
    return out2d.reshape(_SEQ_LEN, -1)


def kernel(et, mp, co, vol, comp_idx, Wq, bq, Wk, bk, Wv, bv, Wo, bo):
    del comp_idx
    et2 = et.reshape(_SEQ_LEN, -1)
    co2 = co.reshape(_SEQ_LEN, -1)
    mp2 = mp.reshape(_SEQ_LEN, -1)
    vol2 = vol.reshape(_SEQ_LEN, -1)
    cc2d = jnp.concatenate([et2, co2, mp2, vol2], axis=-1).reshape(-1, _D_MODEL)
    return _run(cc2d, Wq, bq, Wk, bk, Wv, bv, Wo, bo)
